# baseline (speedup 1.0000x reference)
import functools

import jax
import jax.numpy as jnp
from jax import lax
from jax.experimental import pallas as pl
from jax.experimental.pallas import tpu as pltpu
from jax.experimental.pallas import tpu_sc as plsc

_CB = 8
_NBUF = 2
_LANE = 16


@functools.lru_cache(maxsize=None)
def _fused_call(V, D, B, F, NC, NS):
    NW = NC * NS
    n_e = B // NW
    n_s = n_e // _CB
    nouter = (n_s + _NBUF - 1) // _NBUF
    RP = ((V + NS - 1) // NS + 7) // 8 * 8
    VL = V - (NS - 1) * RP
    assert B % NW == 0 and n_e % _CB == 0 and n_s >= _NBUF
    assert 0 < VL <= RP and VL % 8 == 0 and D % _LANE == 0
    mesh = plsc.VectorSubcoreMesh(core_axis_name="c", subcore_axis_name="s")

    @functools.partial(
        pl.kernel,
        mesh=mesh,
        out_type=jax.ShapeDtypeStruct((B, F, D), jnp.float32),
        scratch_types=(
            [
                pltpu.VMEM_SHARED((NS * RP, D), jnp.float32),
                pltpu.VMEM((n_e, F), jnp.int32),
                pltpu.VMEM((RP, D), jnp.float32),
                pltpu.VMEM((RP, 1), jnp.float32),
                pltpu.VMEM((1, D), jnp.float32),
                pltpu.VMEM((1, D), jnp.float32),
            ]
            + [pltpu.VMEM((_CB, F, D), jnp.float32) for _ in range(_NBUF)]
            + [pltpu.SemaphoreType.DMA]
            + [pltpu.SemaphoreType.DMA for _ in range(2 * _NBUF)]
        ),
    )
    def fused_k(e_hbm, l_hbm, h_hbm, r_hbm, idx_hbm, out_hbm,
                table_sp, idx_v, w_v, r_v, l_v, h_v, *rest):
        bufs = rest[:_NBUF]
        isem = rest[_NBUF]
        gsems = rest[_NBUF + 1:_NBUF + 1 + _NBUF]
        wsems = rest[_NBUF + 1 + _NBUF:]
        cid = lax.axis_index("c")
        sid = lax.axis_index("s")
        wid = sid * NC + cid
        e0 = wid * n_e

        pltpu.async_copy(idx_hbm.at[pl.ds(e0, n_e)], idx_v, isem)

        row0 = sid * RP

        @pl.when(sid < NS - 1)
        def _():
            pltpu.sync_copy(e_hbm.at[pl.ds(row0, RP)], w_v)
            pltpu.sync_copy(r_hbm.at[pl.ds(row0, RP)], r_v)

        @pl.when(sid == NS - 1)
        def _():
            pltpu.sync_copy(e_hbm.at[pl.ds((NS - 1) * RP, VL)],
                            w_v.at[pl.ds(0, VL)])
            pltpu.sync_copy(r_hbm.at[pl.ds((NS - 1) * RP, VL)],
                            r_v.at[pl.ds(0, VL)])
        pltpu.sync_copy(l_hbm, l_v)
        pltpu.sync_copy(h_hbm, h_v)

        def row_body(i, carry):
            r_s = r_v[i, :][0]
            for k in range(D // _LANE):
                d = pl.ds(k * _LANE, _LANE)
                lk = l_v[0, d]
                hk = h_v[0, d]
                w_v[i, d] = r_s * (lk - hk) + hk + w_v[i, d]
            return carry

        lax.fori_loop(0, RP, row_body, 0)
        pltpu.sync_copy(w_v, table_sp.at[pl.ds(row0, RP)])
        plsc.subcore_barrier()

        pltpu.make_async_copy(idx_hbm.at[pl.ds(0, n_e)], idx_v, isem).wait()

        def gathers(s, b):
            for ci in range(_CB):
                pltpu.async_copy(
                    table_sp.at[idx_v.at[s * _CB + ci]], bufs[b].at[ci],
                    gsems[b])

        def drain_write(s, b):
            for ci in range(_CB):
                pltpu.make_async_copy(
                    table_sp.at[idx_v.at[0]], bufs[b].at[ci], gsems[b]).wait()
            pltpu.async_copy(
                bufs[b], out_hbm.at[pl.ds(e0 + s * _CB, _CB)], wsems[b])

        def wait_write(b):
            pltpu.make_async_copy(
                bufs[b], out_hbm.at[pl.ds(0, _CB)], wsems[b]).wait()

        gathers(0, 0)

        def body(i, carry):
            for pos in range(_NBUF):
                s = i * _NBUF + pos + 1
                b = (pos + 1) % _NBUF
                bprev = pos % _NBUF

                @pl.when(jnp.logical_and(s >= _NBUF, s < n_s))
                def _():
                    wait_write(b)

                @pl.when(s < n_s)
                def _():
                    gathers(s, b)

                @pl.when(s - 1 < n_s)
                def _():
                    drain_write(s - 1, bprev)
            return carry

        lax.fori_loop(0, nouter, body, 0)
        for b in range(_NBUF):
            wait_write(b)

    return fused_k


_NCHUNK = 2


def kernel(idx, E, l, h, r):
    B, F = idx.shape
    V, D = E.shape
    info = plsc.get_sparse_core_info()
    NC, NS = info.num_cores, info.num_subcores
    idx = idx.astype(jnp.int32)
    BC = B // _NCHUNK
    call = _fused_call(V, D, BC, F, NC, NS)
    out = jnp.zeros((B, F, D), jnp.float32)
    for i in range(_NCHUNK):
        chunk = call(E, l, h, r, idx[i * BC:(i + 1) * BC])
        out = lax.dynamic_update_slice(out, chunk, (i * BC, 0, 0))
        out = lax.optimization_barrier(out)
    return out


# speedup vs baseline: 1.5822x; 1.5822x over previous
import functools

import jax
import jax.numpy as jnp
from jax import lax
from jax.experimental import pallas as pl
from jax.experimental.pallas import tpu as pltpu
from jax.experimental.pallas import tpu_sc as plsc

_CB = 8
_NBUF = 2
_LANE = 16


@functools.lru_cache(maxsize=None)
def _fused_call(V, D, B, F, NC, NS):
    NW = NC * NS
    n_e = B // NW
    n_s = n_e // _CB
    nouter = (n_s + _NBUF - 1) // _NBUF
    RP = ((V + NS - 1) // NS + 7) // 8 * 8
    VL = V - (NS - 1) * RP
    assert B % NW == 0 and n_e % _CB == 0 and n_s >= _NBUF
    assert 0 < VL <= RP and VL % 8 == 0 and D % _LANE == 0
    mesh = plsc.VectorSubcoreMesh(core_axis_name="c", subcore_axis_name="s")

    @functools.partial(
        pl.kernel,
        mesh=mesh,
        out_type=jax.ShapeDtypeStruct((B, F, D), jnp.float32),
        scratch_types=(
            [
                pltpu.VMEM_SHARED((NS * RP, D), jnp.float32),
                pltpu.VMEM((n_e, F), jnp.int32),
                pltpu.VMEM((RP, D), jnp.float32),
                pltpu.VMEM((RP, 1), jnp.float32),
                pltpu.VMEM((1, D), jnp.float32),
                pltpu.VMEM((1, D), jnp.float32),
            ]
            + [pltpu.VMEM((_CB, F, D), jnp.float32) for _ in range(_NBUF)]
            + [pltpu.SemaphoreType.DMA]
            + [pltpu.SemaphoreType.DMA for _ in range(2 * _NBUF)]
        ),
    )
    def fused_k(e_hbm, l_hbm, h_hbm, r_hbm, idx_hbm, out_hbm,
                table_sp, idx_v, w_v, r_v, l_v, h_v, *rest):
        bufs = rest[:_NBUF]
        isem = rest[_NBUF]
        gsems = rest[_NBUF + 1:_NBUF + 1 + _NBUF]
        wsems = rest[_NBUF + 1 + _NBUF:]
        cid = lax.axis_index("c")
        sid = lax.axis_index("s")
        wid = sid * NC + cid
        e0 = wid * n_e

        pltpu.async_copy(idx_hbm.at[pl.ds(e0, n_e)], idx_v, isem)

        row0 = sid * RP

        @pl.when(sid < NS - 1)
        def _():
            pltpu.sync_copy(e_hbm.at[pl.ds(row0, RP)], w_v)
            pltpu.sync_copy(r_hbm.at[pl.ds(row0, RP)], r_v)

        @pl.when(sid == NS - 1)
        def _():
            pltpu.sync_copy(e_hbm.at[pl.ds((NS - 1) * RP, VL)],
                            w_v.at[pl.ds(0, VL)])
            pltpu.sync_copy(r_hbm.at[pl.ds((NS - 1) * RP, VL)],
                            r_v.at[pl.ds(0, VL)])
        pltpu.sync_copy(l_hbm, l_v)
        pltpu.sync_copy(h_hbm, h_v)

        def row_body(i, carry):
            r_s = r_v[i, :][0]
            for k in range(D // _LANE):
                d = pl.ds(k * _LANE, _LANE)
                lk = l_v[0, d]
                hk = h_v[0, d]
                w_v[i, d] = r_s * (lk - hk) + hk + w_v[i, d]
            return carry

        lax.fori_loop(0, RP, row_body, 0)
        pltpu.sync_copy(w_v, table_sp.at[pl.ds(row0, RP)])
        plsc.subcore_barrier()

        pltpu.make_async_copy(idx_hbm.at[pl.ds(0, n_e)], idx_v, isem).wait()

        def gathers(s, b):
            for ci in range(_CB):
                pltpu.async_copy(
                    table_sp.at[idx_v.at[s * _CB + ci]], bufs[b].at[ci],
                    gsems[b])

        def drain_write(s, b):
            for ci in range(_CB):
                pltpu.make_async_copy(
                    table_sp.at[idx_v.at[0]], bufs[b].at[ci], gsems[b]).wait()
            pltpu.async_copy(
                bufs[b], out_hbm.at[pl.ds(e0 + s * _CB, _CB)], wsems[b])

        def wait_write(b):
            pltpu.make_async_copy(
                bufs[b], out_hbm.at[pl.ds(0, _CB)], wsems[b]).wait()

        gathers(0, 0)

        def body(i, carry):
            for pos in range(_NBUF):
                s = i * _NBUF + pos + 1
                b = (pos + 1) % _NBUF
                bprev = pos % _NBUF

                @pl.when(jnp.logical_and(s >= _NBUF, s < n_s))
                def _():
                    wait_write(b)

                @pl.when(s < n_s)
                def _():
                    gathers(s, b)

                @pl.when(s - 1 < n_s)
                def _():
                    drain_write(s - 1, bprev)
            return carry

        lax.fori_loop(0, nouter, body, 0)
        for b in range(_NBUF):
            wait_write(b)

    return fused_k


def kernel(idx, E, l, h, r):
    B, F = idx.shape
    V, D = E.shape
    info = plsc.get_sparse_core_info()
    NC, NS = info.num_cores, info.num_subcores
    return _fused_call(V, D, B, F, NC, NS)(
        E, l, h, r, idx.astype(jnp.int32))
